# Initial kernel scaffold; baseline (speedup 1.0000x reference)
import jax
import jax.numpy as jnp
from jax.experimental import pallas as pl


def kernel(x, edge_attr, A1_0, b1_0, A2_0, b2_0, root0, bias0, bn_g0, bn_b0, A1_1, b1_1, A2_1, b2_1, root1, bias1, bn_g1, bn_b1, gate_w, gate_b, cls_w1, cls_b1, cls_w2, cls_b2, reg_w1, reg_b1, reg_w2, reg_b2, edge_index, batch):
    raise NotImplementedError("write your pallas kernel here")


# speedup vs baseline: 2.5255x; 2.5255x over previous
import functools

import jax
import jax.numpy as jnp
from jax import lax
from jax.experimental import pallas as pl
from jax.experimental.pallas import tpu as pltpu
from jax.experimental.pallas import tpu_sc as plsc

_N = 10000
_E = 160000
_IN = 128
_H = 8
_G = 32
_NC = 2
_NS = 16
_NW = _NC * _NS
_PW = _E // _NW
_CW = 125
_CH = _PW // _CW
_NP = 10112
_RPS = _NP // _NS

_SC_PARAMS = pltpu.CompilerParams(use_tc_tiling_on_sc=False)

_f32 = jnp.float32
_bf16 = jnp.bfloat16


def _gather(table, idx3, d):
    mesh = plsc.VectorSubcoreMesh(core_axis_name="c", subcore_axis_name="s")

    @functools.partial(
        pl.kernel,
        out_type=jax.ShapeDtypeStruct((_E, d), _f32),
        mesh=mesh,
        compiler_params=_SC_PARAMS,
        scratch_types=[
            pltpu.VMEM((_CH, _CW), jnp.int32),
            pltpu.VMEM((2, _CW, d), _f32),
            pltpu.SemaphoreType.DMA,
            pltpu.SemaphoreType.DMA,
        ],
    )
    def k(table_hbm, idx_hbm, out_hbm, idx_v, rows_v, gsem, ssem):
        wid = lax.axis_index("s") * _NC + lax.axis_index("c")
        pltpu.sync_copy(idx_hbm.at[wid], idx_v)
        pltpu.async_copy(table_hbm.at[idx_v.at[0]], rows_v.at[0], gsem)

        @pl.loop(0, _CH)
        def _(j):
            slot = lax.rem(j, 2)
            nxt = lax.rem(j + 1, 2)
            pltpu.make_async_copy(table_hbm.at[pl.ds(0, _CW)],
                                  rows_v.at[slot], gsem).wait()

            @pl.when(j > 0)
            def _():
                pltpu.make_async_copy(rows_v.at[nxt],
                                      out_hbm.at[pl.ds(0, _CW)], ssem).wait()

            @pl.when(j + 1 < _CH)
            def _():
                pltpu.async_copy(table_hbm.at[idx_v.at[j + 1]],
                                 rows_v.at[nxt], gsem)

            pltpu.async_copy(
                rows_v.at[slot],
                out_hbm.at[pl.ds(wid * _PW + j * _CW, _CW)], ssem)

        pltpu.make_async_copy(rows_v.at[0],
                              out_hbm.at[pl.ds(0, _CW)], ssem).wait()

    return k(table, idx3)


def _scatter(vals, idx3, zeros16):
    mesh = plsc.VectorSubcoreMesh(core_axis_name="c", subcore_axis_name="s")

    @functools.partial(
        pl.kernel,
        out_type=jax.ShapeDtypeStruct((2 * _NP, 16), _f32),
        mesh=mesh,
        compiler_params=_SC_PARAMS,
        scratch_types=[
            pltpu.VMEM((_CH, _CW), jnp.int32),
            pltpu.VMEM((_PW, 16), _f32),
            pltpu.VMEM_SHARED((_NP, 16), _f32),
            pltpu.SemaphoreType.DMA,
        ],
    )
    def k(vals_hbm, idx_hbm, zeros_hbm, out_hbm, idx_v, vals_v, acc_sh, sem):
        cid = lax.axis_index("c")
        sid = lax.axis_index("s")
        wid = sid * _NC + cid
        pltpu.sync_copy(zeros_hbm.at[pl.ds(sid * _RPS, _RPS)],
                        acc_sh.at[pl.ds(sid * _RPS, _RPS)])
        pltpu.sync_copy(idx_hbm.at[wid], idx_v)
        pltpu.sync_copy(vals_hbm.at[pl.ds(wid * _PW, _PW)], vals_v)
        plsc.subcore_barrier()

        @pl.loop(0, _CH)
        def _(j):
            pltpu.sync_copy(vals_v.at[pl.ds(j * _CW, _CW)],
                            acc_sh.at[idx_v.at[j]], add=True)

        plsc.subcore_barrier()
        pltpu.sync_copy(acc_sh.at[pl.ds(sid * _RPS, _RPS)],
                        out_hbm.at[pl.ds(cid * _NP + sid * _RPS, _RPS)])

    return k(vals, idx3, zeros16)


_TE = 1280
_GE = _E // _TE

def _edge_body(ea_ref, xs_ref, a1_ref, b1_ref, a2_ref, b2_ref,
               a11_ref, b11_ref, a21_ref, b21_ref, r_ref, s_ref,
               msg_ref, w1_ref):
    ea = ea_ref[...].astype(_bf16)
    h = jnp.maximum(
        jnp.dot(ea, a1_ref[...].astype(_bf16), preferred_element_type=_f32)
        + b1_ref[...], 0.0)
    w = jnp.dot(h.astype(_bf16), a2_ref[...].astype(_bf16),
                preferred_element_type=_f32) + b2_ref[...]
    xr = jnp.dot(xs_ref[...].astype(_bf16), r_ref[...],
                 preferred_element_type=_f32)
    msg = jnp.dot(xr * w, s_ref[...], preferred_element_type=_f32)
    col = lax.broadcasted_iota(jnp.int32, (_TE, _H), 1)
    oz = jnp.where(col == 0, 1.0, 0.0).astype(_f32)
    msg_ref[...] = jnp.concatenate([msg, oz], axis=1)

    h1 = jnp.maximum(
        jnp.dot(ea, a11_ref[...].astype(_bf16), preferred_element_type=_f32)
        + b11_ref[...], 0.0)
    w1_ref[...] = jnp.dot(h1.astype(_bf16), a21_ref[...].astype(_bf16),
                          preferred_element_type=_f32) + b21_ref[...]


def _edge_call(ea, xs, a1, b1, a2, b2, a11, b11, a21, b21, r, s):
    hw = _IN * _H
    hh = _H * _H
    return pl.pallas_call(
        _edge_body,
        grid=(_GE,),
        in_specs=[
            pl.BlockSpec((_TE, 16), lambda i: (i, 0)),
            pl.BlockSpec((_TE, _IN), lambda i: (i, 0)),
            pl.BlockSpec((16, hw), lambda i: (0, 0)),
            pl.BlockSpec((1, hw), lambda i: (0, 0)),
            pl.BlockSpec((hw, hw), lambda i: (0, 0)),
            pl.BlockSpec((1, hw), lambda i: (0, 0)),
            pl.BlockSpec((16, hh), lambda i: (0, 0)),
            pl.BlockSpec((1, hh), lambda i: (0, 0)),
            pl.BlockSpec((hh, hh), lambda i: (0, 0)),
            pl.BlockSpec((1, hh), lambda i: (0, 0)),
            pl.BlockSpec((_IN, hw), lambda i: (0, 0)),
            pl.BlockSpec((hw, _H), lambda i: (0, 0)),
        ],
        out_specs=[
            pl.BlockSpec((_TE, 16), lambda i: (i, 0)),
            pl.BlockSpec((_TE, hh), lambda i: (i, 0)),
        ],
        out_shape=[
            jax.ShapeDtypeStruct((_E, 16), _f32),
            jax.ShapeDtypeStruct((_E, hh), _f32),
        ],
    )(ea, xs, a1, b1, a2, b2, a11, b11, a21, b21, r, s)


def _agg_bn(part, cnt_col, root_w, hin, bias, g, b):
    s = part[0:_N, 0:_H] + part[_NP:_NP + _N, 0:_H]
    cnt = part[0:_N, _H:_H + 1] + part[_NP:_NP + _N, _H:_H + 1]
    agg = s / jnp.maximum(cnt, 1.0)
    h0 = agg + jnp.dot(hin, root_w, preferred_element_type=_f32) + bias
    m = jnp.mean(h0, axis=0, keepdims=True)
    v = jnp.mean((h0 - m) ** 2, axis=0, keepdims=True)
    return jnp.maximum((h0 - m) * lax.rsqrt(v + 1e-5) * g + b, 0.0)


def _node0_body(part_ref, x_ref, root_ref, bias_ref, g_ref, b_ref, out_ref):
    h = _agg_bn(part_ref[...], None, root_ref[...], x_ref[...],
                bias_ref[...], g_ref[...], b_ref[...])
    out_ref[...] = jnp.concatenate([h, jnp.zeros_like(h)], axis=1)


def _node0_call(part, x, root_w, bias, g, b):
    return pl.pallas_call(
        _node0_body,
        grid=(1,),
        in_specs=[
            pl.BlockSpec((2 * _NP, 16), lambda i: (0, 0)),
            pl.BlockSpec((_N, _IN), lambda i: (0, 0)),
            pl.BlockSpec((_IN, _H), lambda i: (0, 0)),
            pl.BlockSpec((1, _H), lambda i: (0, 0)),
            pl.BlockSpec((1, _H), lambda i: (0, 0)),
            pl.BlockSpec((1, _H), lambda i: (0, 0)),
        ],
        out_specs=pl.BlockSpec((_N, 16), lambda i: (0, 0)),
        out_shape=jax.ShapeDtypeStruct((_N, 16), _f32),
    )(part, x, root_w, bias, g, b)


_TM = 6400
_GM = _E // _TM

def _msg1_body(hs_ref, w1_ref, r8_ref, s8_ref, out_ref):
    hs = hs_ref[:, 0:_H]
    hr = jnp.dot(hs, r8_ref[...], preferred_element_type=_f32)
    msg = jnp.dot(hr * w1_ref[...], s8_ref[...], preferred_element_type=_f32)
    col = lax.broadcasted_iota(jnp.int32, (_TM, _H), 1)
    oz = jnp.where(col == 0, 1.0, 0.0).astype(_f32)
    out_ref[...] = jnp.concatenate([msg, oz], axis=1)


def _msg1_call(hs, w1, r8, s8):
    hh = _H * _H
    return pl.pallas_call(
        _msg1_body,
        grid=(_GM,),
        in_specs=[
            pl.BlockSpec((_TM, 16), lambda i: (i, 0)),
            pl.BlockSpec((_TM, hh), lambda i: (i, 0)),
            pl.BlockSpec((_H, hh), lambda i: (0, 0)),
            pl.BlockSpec((hh, _H), lambda i: (0, 0)),
        ],
        out_specs=pl.BlockSpec((_TM, 16), lambda i: (i, 0)),
        out_shape=jax.ShapeDtypeStruct((_E, 16), _f32),
    )(hs, w1, r8, s8)


def _final_body(part_ref, h_ref, root_ref, bias_ref, g_ref, b_ref,
                gw_ref, gb_ref, cw1_ref, cb1_ref, cw2_ref, cb2_ref,
                rw1_ref, rb1_ref, rw2_ref, rb2_ref, batch_ref,
                cls_ref, reg_ref):
    z = _agg_bn(part_ref[...], None, root_ref[...], h_ref[:, 0:_H],
                bias_ref[...], g_ref[...], b_ref[...])
    gate = jnp.dot(z, gw_ref[...], preferred_element_type=_f32) + gb_ref[...]
    gids = lax.broadcasted_iota(jnp.int32, (1, _G), 1)
    maskb = batch_ref[...] == gids
    maskf = maskb.astype(_f32)
    gmax = jnp.max(jnp.where(maskb, gate, -jnp.inf), axis=0, keepdims=True)
    gmax = jnp.where(jnp.isfinite(gmax), gmax, 0.0)
    gmax_n = jnp.sum(maskf * gmax, axis=1, keepdims=True)
    a = jnp.exp(gate - gmax_n)
    denom = jnp.sum(maskf * a, axis=0, keepdims=True)
    denom_n = jnp.sum(maskf * denom, axis=1, keepdims=True)
    alpha = a / (denom_n + 1e-16)
    gpool = lax.dot_general(maskf, alpha * z, (((0,), (0,)), ((), ())),
                            preferred_element_type=_f32)
    c1 = jnp.maximum(
        jnp.dot(gpool, cw1_ref[...], preferred_element_type=_f32)
        + cb1_ref[...], 0.0)
    cls_ref[...] = jnp.dot(c1, cw2_ref[...],
                           preferred_element_type=_f32) + cb2_ref[...]
    r1 = jnp.maximum(
        jnp.dot(gpool, rw1_ref[...], preferred_element_type=_f32)
        + rb1_ref[...], 0.0)
    reg_ref[...] = jnp.dot(r1, rw2_ref[...],
                           preferred_element_type=_f32) + rb2_ref[...]


def _final_call(part, h16, root_w, bias, g, b, gw, gb,
                cw1, cb1, cw2, cb2, rw1, rb1, rw2, rb2, batch_col):
    full = lambda r, c: pl.BlockSpec((r, c), lambda i: (0, 0))
    return pl.pallas_call(
        _final_body,
        grid=(1,),
        in_specs=[
            full(2 * _NP, 16),
            full(_N, 16),
            full(_H, _H), full(1, _H), full(1, _H), full(1, _H),
            full(_H, 1), full(1, 1),
            full(_H, _H), full(1, _H), full(_H, 9), full(1, 9),
            full(_H, _H), full(1, _H), full(_H, 1), full(1, 1),
            full(_N, 1),
        ],
        out_specs=[full(_G, 9), full(_G, 1)],
        out_shape=[
            jax.ShapeDtypeStruct((_G, 9), _f32),
            jax.ShapeDtypeStruct((_G, 1), _f32),
        ],
    )(part, h16, root_w, bias, g, b, gw, gb,
      cw1, cb1, cw2, cb2, rw1, rb1, rw2, rb2, batch_col)


def kernel(x, edge_attr, A1_0, b1_0, A2_0, b2_0, root0, bias0, bn_g0, bn_b0,
           A1_1, b1_1, A2_1, b2_1, root1, bias1, bn_g1, bn_b1,
           gate_w, gate_b, cls_w1, cls_b1, cls_w2, cls_b2,
           reg_w1, reg_b1, reg_w2, reg_b2, edge_index, batch):
    row = lambda t: t.reshape(1, -1)
    src3 = edge_index[0].reshape(_NW, _CH, _CW)
    dst3 = edge_index[1].reshape(_NW, _CH, _CW)
    zeros16 = jnp.zeros((_NP, 16), _f32)
    r = jnp.repeat(jnp.eye(_IN, dtype=_bf16), _H, axis=1)
    s = jnp.tile(jnp.eye(_H, dtype=_f32), (_IN, 1))
    r8 = jnp.repeat(jnp.eye(_H, dtype=_f32), _H, axis=1)
    s8 = jnp.tile(jnp.eye(_H, dtype=_f32), (_H, 1))

    xs = _gather(x, src3, _IN)
    msg0, w1 = _edge_call(edge_attr, xs, A1_0, row(b1_0), A2_0, row(b2_0),
                          A1_1, row(b1_1), A2_1, row(b2_1), r, s)
    part0 = _scatter(msg0, dst3, zeros16)
    h16 = _node0_call(part0, x, root0, row(bias0), row(bn_g0), row(bn_b0))
    hs = _gather(h16, src3, 16)
    msg1 = _msg1_call(hs, w1, r8, s8)
    part1 = _scatter(msg1, dst3, zeros16)
    cls, reg = _final_call(part1, h16, root1, row(bias1), row(bn_g1),
                           row(bn_b1), gate_w, row(gate_b),
                           cls_w1, row(cls_b1), cls_w2, row(cls_b2),
                           reg_w1, row(reg_b1), reg_w2, row(reg_b2),
                           batch.reshape(-1, 1))
    return (cls, reg)


# speedup vs baseline: 2.6396x; 1.0452x over previous
import functools

import jax
import jax.numpy as jnp
from jax import lax
from jax.experimental import pallas as pl
from jax.experimental.pallas import tpu as pltpu
from jax.experimental.pallas import tpu_sc as plsc

_N = 10000
_E = 160000
_IN = 128
_H = 8
_G = 32
_NC = 2
_NS = 16
_NW = _NC * _NS
_PW = _E // _NW
_CW = 125
_CH = _PW // _CW
_NP = 10112
_RPS = _NP // _NS

_SC_PARAMS = pltpu.CompilerParams(use_tc_tiling_on_sc=False)

_f32 = jnp.float32
_bf16 = jnp.bfloat16


def _gather(table, idx3, d):
    mesh = plsc.VectorSubcoreMesh(core_axis_name="c", subcore_axis_name="s")

    @functools.partial(
        pl.kernel,
        out_type=jax.ShapeDtypeStruct((_E, d), _f32),
        mesh=mesh,
        compiler_params=_SC_PARAMS,
        scratch_types=[
            pltpu.VMEM((_CH, _CW), jnp.int32),
            pltpu.VMEM((2, _CW, d), _f32),
            pltpu.SemaphoreType.DMA,
            pltpu.SemaphoreType.DMA,
        ],
    )
    def k(table_hbm, idx_hbm, out_hbm, idx_v, rows_v, gsem, ssem):
        wid = lax.axis_index("s") * _NC + lax.axis_index("c")
        pltpu.sync_copy(idx_hbm.at[wid], idx_v)
        pltpu.async_copy(table_hbm.at[idx_v.at[0]], rows_v.at[0], gsem)

        @pl.loop(0, _CH)
        def _(j):
            slot = lax.rem(j, 2)
            nxt = lax.rem(j + 1, 2)
            pltpu.make_async_copy(table_hbm.at[pl.ds(0, _CW)],
                                  rows_v.at[slot], gsem).wait()

            @pl.when(j > 0)
            def _():
                pltpu.make_async_copy(rows_v.at[nxt],
                                      out_hbm.at[pl.ds(0, _CW)], ssem).wait()

            @pl.when(j + 1 < _CH)
            def _():
                pltpu.async_copy(table_hbm.at[idx_v.at[j + 1]],
                                 rows_v.at[nxt], gsem)

            pltpu.async_copy(
                rows_v.at[slot],
                out_hbm.at[pl.ds(wid * _PW + j * _CW, _CW)], ssem)

        pltpu.make_async_copy(rows_v.at[0],
                              out_hbm.at[pl.ds(0, _CW)], ssem).wait()

    return k(table, idx3)


def _scatter(vals, idx3, zeros16):
    mesh = plsc.VectorSubcoreMesh(core_axis_name="c", subcore_axis_name="s")

    @functools.partial(
        pl.kernel,
        out_type=jax.ShapeDtypeStruct((2 * _NP, 16), _f32),
        mesh=mesh,
        compiler_params=_SC_PARAMS,
        scratch_types=[
            pltpu.VMEM((_CH, _CW), jnp.int32),
            pltpu.VMEM((_PW, 16), _f32),
            pltpu.VMEM_SHARED((_NP, 16), _f32),
            pltpu.SemaphoreType.DMA,
        ],
    )
    def k(vals_hbm, idx_hbm, zeros_hbm, out_hbm, idx_v, vals_v, acc_sh, sem):
        cid = lax.axis_index("c")
        sid = lax.axis_index("s")
        wid = sid * _NC + cid
        pltpu.sync_copy(zeros_hbm.at[pl.ds(sid * _RPS, _RPS)],
                        acc_sh.at[pl.ds(sid * _RPS, _RPS)])
        pltpu.sync_copy(idx_hbm.at[wid], idx_v)
        pltpu.sync_copy(vals_hbm.at[pl.ds(wid * _PW, _PW)], vals_v)
        plsc.subcore_barrier()

        @pl.loop(0, _CH)
        def _(j):
            pltpu.sync_copy(vals_v.at[pl.ds(j * _CW, _CW)],
                            acc_sh.at[idx_v.at[j]], add=True)

        plsc.subcore_barrier()
        pltpu.sync_copy(acc_sh.at[pl.ds(sid * _RPS, _RPS)],
                        out_hbm.at[pl.ds(cid * _NP + sid * _RPS, _RPS)])

    return k(vals, idx3, zeros16)


_TE = 1280
_GE = _E // _TE

def _edge_body(ea_ref, xs_ref, a1_ref, b1_ref, a2_ref, b2_ref,
               a11_ref, b11_ref, a21_ref, b21_ref, r_ref,
               msg_ref, w1_ref):
    ea = ea_ref[...].astype(_bf16)
    h = jnp.maximum(
        jnp.dot(ea, a1_ref[...].astype(_bf16), preferred_element_type=_f32)
        + b1_ref[...], 0.0).astype(_bf16)
    w = jnp.dot(h, a2_ref[...].astype(_bf16),
                preferred_element_type=_f32) + b2_ref[...]
    xr = jnp.dot(xs_ref[...].astype(_bf16), r_ref[...],
                 preferred_element_type=_f32)
    p = xr * w
    while p.shape[1] > _H:
        half = p.shape[1] // 2
        p = p[:, :half] + p[:, half:]
    col = lax.broadcasted_iota(jnp.int32, (_TE, _H), 1)
    oz = jnp.where(col == 0, 1.0, 0.0).astype(_f32)
    msg_ref[...] = jnp.concatenate([p, oz], axis=1)

    h1 = jnp.maximum(
        jnp.dot(ea, a11_ref[...].astype(_bf16), preferred_element_type=_f32)
        + b11_ref[...], 0.0).astype(_bf16)
    w1_ref[...] = jnp.dot(h1, a21_ref[...].astype(_bf16),
                          preferred_element_type=_f32) + b21_ref[...]


def _edge_call(ea, xs, a1, b1, a2, b2, a11, b11, a21, b21, r):
    hw = _IN * _H
    hh = _H * _H
    return pl.pallas_call(
        _edge_body,
        grid=(_GE,),
        in_specs=[
            pl.BlockSpec((_TE, 16), lambda i: (i, 0)),
            pl.BlockSpec((_TE, _IN), lambda i: (i, 0)),
            pl.BlockSpec((16, hw), lambda i: (0, 0)),
            pl.BlockSpec((1, hw), lambda i: (0, 0)),
            pl.BlockSpec((hw, hw), lambda i: (0, 0)),
            pl.BlockSpec((1, hw), lambda i: (0, 0)),
            pl.BlockSpec((16, hh), lambda i: (0, 0)),
            pl.BlockSpec((1, hh), lambda i: (0, 0)),
            pl.BlockSpec((hh, hh), lambda i: (0, 0)),
            pl.BlockSpec((1, hh), lambda i: (0, 0)),
            pl.BlockSpec((_IN, hw), lambda i: (0, 0)),
        ],
        out_specs=[
            pl.BlockSpec((_TE, 16), lambda i: (i, 0)),
            pl.BlockSpec((_TE, hh), lambda i: (i, 0)),
        ],
        out_shape=[
            jax.ShapeDtypeStruct((_E, 16), _f32),
            jax.ShapeDtypeStruct((_E, hh), _f32),
        ],
    )(ea, xs, a1, b1, a2, b2, a11, b11, a21, b21, r)


def _agg_bn(part, cnt_col, root_w, hin, bias, g, b):
    s = part[0:_N, 0:_H] + part[_NP:_NP + _N, 0:_H]
    cnt = part[0:_N, _H:_H + 1] + part[_NP:_NP + _N, _H:_H + 1]
    agg = s / jnp.maximum(cnt, 1.0)
    h0 = agg + jnp.dot(hin, root_w, preferred_element_type=_f32) + bias
    m = jnp.mean(h0, axis=0, keepdims=True)
    v = jnp.mean((h0 - m) ** 2, axis=0, keepdims=True)
    return jnp.maximum((h0 - m) * lax.rsqrt(v + 1e-5) * g + b, 0.0)


def _node0_body(part_ref, x_ref, root_ref, bias_ref, g_ref, b_ref, out_ref):
    h = _agg_bn(part_ref[...], None, root_ref[...], x_ref[...],
                bias_ref[...], g_ref[...], b_ref[...])
    out_ref[...] = jnp.concatenate([h, jnp.zeros_like(h)], axis=1)


def _node0_call(part, x, root_w, bias, g, b):
    return pl.pallas_call(
        _node0_body,
        grid=(1,),
        in_specs=[
            pl.BlockSpec((2 * _NP, 16), lambda i: (0, 0)),
            pl.BlockSpec((_N, _IN), lambda i: (0, 0)),
            pl.BlockSpec((_IN, _H), lambda i: (0, 0)),
            pl.BlockSpec((1, _H), lambda i: (0, 0)),
            pl.BlockSpec((1, _H), lambda i: (0, 0)),
            pl.BlockSpec((1, _H), lambda i: (0, 0)),
        ],
        out_specs=pl.BlockSpec((_N, 16), lambda i: (0, 0)),
        out_shape=jax.ShapeDtypeStruct((_N, 16), _f32),
    )(part, x, root_w, bias, g, b)


_TM = 6400
_GM = _E // _TM

def _msg1_body(hs_ref, w1_ref, r8_ref, s8_ref, out_ref):
    hs = hs_ref[:, 0:_H]
    hr = jnp.dot(hs, r8_ref[...], preferred_element_type=_f32)
    msg = jnp.dot(hr * w1_ref[...], s8_ref[...], preferred_element_type=_f32)
    col = lax.broadcasted_iota(jnp.int32, (_TM, _H), 1)
    oz = jnp.where(col == 0, 1.0, 0.0).astype(_f32)
    out_ref[...] = jnp.concatenate([msg, oz], axis=1)


def _msg1_call(hs, w1, r8, s8):
    hh = _H * _H
    return pl.pallas_call(
        _msg1_body,
        grid=(_GM,),
        in_specs=[
            pl.BlockSpec((_TM, 16), lambda i: (i, 0)),
            pl.BlockSpec((_TM, hh), lambda i: (i, 0)),
            pl.BlockSpec((_H, hh), lambda i: (0, 0)),
            pl.BlockSpec((hh, _H), lambda i: (0, 0)),
        ],
        out_specs=pl.BlockSpec((_TM, 16), lambda i: (i, 0)),
        out_shape=jax.ShapeDtypeStruct((_E, 16), _f32),
    )(hs, w1, r8, s8)


def _final_body(part_ref, h_ref, root_ref, bias_ref, g_ref, b_ref,
                gw_ref, gb_ref, cw1_ref, cb1_ref, cw2_ref, cb2_ref,
                rw1_ref, rb1_ref, rw2_ref, rb2_ref, batch_ref,
                cls_ref, reg_ref):
    z = _agg_bn(part_ref[...], None, root_ref[...], h_ref[:, 0:_H],
                bias_ref[...], g_ref[...], b_ref[...])
    gate = jnp.dot(z, gw_ref[...], preferred_element_type=_f32) + gb_ref[...]
    gids = lax.broadcasted_iota(jnp.int32, (1, _G), 1)
    maskb = batch_ref[...] == gids
    maskf = maskb.astype(_f32)
    gmax = jnp.max(jnp.where(maskb, gate, -jnp.inf), axis=0, keepdims=True)
    gmax = jnp.where(jnp.isfinite(gmax), gmax, 0.0)
    gmax_n = jnp.sum(maskf * gmax, axis=1, keepdims=True)
    a = jnp.exp(gate - gmax_n)
    denom = jnp.sum(maskf * a, axis=0, keepdims=True)
    denom_n = jnp.sum(maskf * denom, axis=1, keepdims=True)
    alpha = a / (denom_n + 1e-16)
    gpool = lax.dot_general(maskf, alpha * z, (((0,), (0,)), ((), ())),
                            preferred_element_type=_f32)
    c1 = jnp.maximum(
        jnp.dot(gpool, cw1_ref[...], preferred_element_type=_f32)
        + cb1_ref[...], 0.0)
    cls_ref[...] = jnp.dot(c1, cw2_ref[...],
                           preferred_element_type=_f32) + cb2_ref[...]
    r1 = jnp.maximum(
        jnp.dot(gpool, rw1_ref[...], preferred_element_type=_f32)
        + rb1_ref[...], 0.0)
    reg_ref[...] = jnp.dot(r1, rw2_ref[...],
                           preferred_element_type=_f32) + rb2_ref[...]


def _final_call(part, h16, root_w, bias, g, b, gw, gb,
                cw1, cb1, cw2, cb2, rw1, rb1, rw2, rb2, batch_col):
    full = lambda r, c: pl.BlockSpec((r, c), lambda i: (0, 0))
    return pl.pallas_call(
        _final_body,
        grid=(1,),
        in_specs=[
            full(2 * _NP, 16),
            full(_N, 16),
            full(_H, _H), full(1, _H), full(1, _H), full(1, _H),
            full(_H, 1), full(1, 1),
            full(_H, _H), full(1, _H), full(_H, 9), full(1, 9),
            full(_H, _H), full(1, _H), full(_H, 1), full(1, 1),
            full(_N, 1),
        ],
        out_specs=[full(_G, 9), full(_G, 1)],
        out_shape=[
            jax.ShapeDtypeStruct((_G, 9), _f32),
            jax.ShapeDtypeStruct((_G, 1), _f32),
        ],
    )(part, h16, root_w, bias, g, b, gw, gb,
      cw1, cb1, cw2, cb2, rw1, rb1, rw2, rb2, batch_col)


def kernel(x, edge_attr, A1_0, b1_0, A2_0, b2_0, root0, bias0, bn_g0, bn_b0,
           A1_1, b1_1, A2_1, b2_1, root1, bias1, bn_g1, bn_b1,
           gate_w, gate_b, cls_w1, cls_b1, cls_w2, cls_b2,
           reg_w1, reg_b1, reg_w2, reg_b2, edge_index, batch):
    row = lambda t: t.reshape(1, -1)
    src3 = edge_index[0].reshape(_NW, _CH, _CW)
    dst3 = edge_index[1].reshape(_NW, _CH, _CW)
    zeros16 = jnp.zeros((_NP, 16), _f32)
    r = jnp.repeat(jnp.eye(_IN, dtype=_bf16), _H, axis=1)
    r8 = jnp.repeat(jnp.eye(_H, dtype=_f32), _H, axis=1)
    s8 = jnp.tile(jnp.eye(_H, dtype=_f32), (_H, 1))

    xs = _gather(x, src3, _IN)
    msg0, w1 = _edge_call(edge_attr, xs, A1_0, row(b1_0), A2_0, row(b2_0),
                          A1_1, row(b1_1), A2_1, row(b2_1), r)
    part0 = _scatter(msg0, dst3, zeros16)
    h16 = _node0_call(part0, x, root0, row(bias0), row(bn_g0), row(bn_b0))
    hs = _gather(h16, src3, 16)
    msg1 = _msg1_call(hs, w1, r8, s8)
    part1 = _scatter(msg1, dst3, zeros16)
    cls, reg = _final_call(part1, h16, root1, row(bias1), row(bn_g1),
                           row(bn_b1), gate_w, row(gate_b),
                           cls_w1, row(cls_b1), cls_w2, row(cls_b2),
                           reg_w1, row(reg_b1), reg_w2, row(reg_b2),
                           batch.reshape(-1, 1))
    return (cls, reg)


# speedup vs baseline: 2.6446x; 1.0019x over previous
import functools

import jax
import jax.numpy as jnp
from jax import lax
from jax.experimental import pallas as pl
from jax.experimental.pallas import tpu as pltpu
from jax.experimental.pallas import tpu_sc as plsc

_N = 10000
_E = 160000
_IN = 128
_H = 8
_G = 32
_NC = 2
_NS = 16
_NW = _NC * _NS
_PW = _E // _NW
_CW = 125
_CH = _PW // _CW
_NP = 10112
_RPS = _NP // _NS

_SC_PARAMS = pltpu.CompilerParams(use_tc_tiling_on_sc=False)

_f32 = jnp.float32
_bf16 = jnp.bfloat16


def _gather(table, idx3, d):
    mesh = plsc.VectorSubcoreMesh(core_axis_name="c", subcore_axis_name="s")

    @functools.partial(
        pl.kernel,
        out_type=jax.ShapeDtypeStruct((_E, d), _f32),
        mesh=mesh,
        compiler_params=_SC_PARAMS,
        scratch_types=[
            pltpu.VMEM((_CH, _CW), jnp.int32),
            pltpu.VMEM((2, _CW, d), _f32),
            pltpu.SemaphoreType.DMA,
            pltpu.SemaphoreType.DMA,
        ],
    )
    def k(table_hbm, idx_hbm, out_hbm, idx_v, rows_v, gsem, ssem):
        wid = lax.axis_index("s") * _NC + lax.axis_index("c")
        pltpu.sync_copy(idx_hbm.at[wid], idx_v)
        pltpu.async_copy(table_hbm.at[idx_v.at[0]], rows_v.at[0], gsem)

        @pl.loop(0, _CH)
        def _(j):
            slot = lax.rem(j, 2)
            nxt = lax.rem(j + 1, 2)
            pltpu.make_async_copy(table_hbm.at[pl.ds(0, _CW)],
                                  rows_v.at[slot], gsem).wait()

            @pl.when(j > 0)
            def _():
                pltpu.make_async_copy(rows_v.at[nxt],
                                      out_hbm.at[pl.ds(0, _CW)], ssem).wait()

            @pl.when(j + 1 < _CH)
            def _():
                pltpu.async_copy(table_hbm.at[idx_v.at[j + 1]],
                                 rows_v.at[nxt], gsem)

            pltpu.async_copy(
                rows_v.at[slot],
                out_hbm.at[pl.ds(wid * _PW + j * _CW, _CW)], ssem)

        pltpu.make_async_copy(rows_v.at[0],
                              out_hbm.at[pl.ds(0, _CW)], ssem).wait()

    return k(table, idx3)


def _scatter(vals, idx3, zeros16):
    mesh = plsc.VectorSubcoreMesh(core_axis_name="c", subcore_axis_name="s")

    @functools.partial(
        pl.kernel,
        out_type=jax.ShapeDtypeStruct((2 * _NP, 16), _f32),
        mesh=mesh,
        compiler_params=_SC_PARAMS,
        scratch_types=[
            pltpu.VMEM((_CH, _CW), jnp.int32),
            pltpu.VMEM((_PW, 16), _f32),
            pltpu.VMEM_SHARED((_NP, 16), _f32),
            pltpu.SemaphoreType.DMA,
        ],
    )
    def k(vals_hbm, idx_hbm, zeros_hbm, out_hbm, idx_v, vals_v, acc_sh, sem):
        cid = lax.axis_index("c")
        sid = lax.axis_index("s")
        wid = sid * _NC + cid
        pltpu.sync_copy(zeros_hbm.at[pl.ds(sid * _RPS, _RPS)],
                        acc_sh.at[pl.ds(sid * _RPS, _RPS)])
        pltpu.sync_copy(idx_hbm.at[wid], idx_v)
        pltpu.sync_copy(vals_hbm.at[pl.ds(wid * _PW, _PW)], vals_v)
        plsc.subcore_barrier()

        @pl.loop(0, _CH)
        def _(j):
            pltpu.sync_copy(vals_v.at[pl.ds(j * _CW, _CW)],
                            acc_sh.at[idx_v.at[j]], add=True)

        plsc.subcore_barrier()
        pltpu.sync_copy(acc_sh.at[pl.ds(sid * _RPS, _RPS)],
                        out_hbm.at[pl.ds(cid * _NP + sid * _RPS, _RPS)])

    return k(vals, idx3, zeros16)


_TE = 1280
_GE = _E // _TE

_TDN = (((0,), (0,)), ((), ()))


def _edge_body(ea_ref, xs_ref, a1_ref, b1_ref, a2_ref, b2_ref,
               a11_ref, b11_ref, a21_ref, b21_ref,
               msg_ref, w1_ref):
    ea = ea_ref[...].astype(_bf16)
    h = jnp.maximum(
        lax.dot_general(ea, a1_ref[...].astype(_bf16), _TDN,
                        preferred_element_type=_f32)
        + b1_ref[...], 0.0).astype(_bf16)
    w = jnp.dot(h, a2_ref[...].astype(_bf16),
                preferred_element_type=_f32) + b2_ref[...]
    idx = lax.broadcasted_iota(jnp.int32, (_TE, _IN * _H), 1) // _H
    xr = jnp.take_along_axis(xs_ref[...], idx, axis=1)
    p = xr * w
    while p.shape[1] > _H:
        half = p.shape[1] // 2
        p = p[:, :half] + p[:, half:]
    col = lax.broadcasted_iota(jnp.int32, (_TE, _H), 1)
    oz = jnp.where(col == 0, 1.0, 0.0).astype(_f32)
    msg_ref[...] = jnp.concatenate([p, oz], axis=1)

    h1 = jnp.maximum(
        lax.dot_general(ea, a11_ref[...].astype(_bf16), _TDN,
                        preferred_element_type=_f32)
        + b11_ref[...], 0.0).astype(_bf16)
    w1_ref[...] = jnp.dot(h1, a21_ref[...].astype(_bf16),
                          preferred_element_type=_f32) + b21_ref[...]


def _edge_call(ea, xs, a1, b1, a2, b2, a11, b11, a21, b21):
    hw = _IN * _H
    hh = _H * _H
    return pl.pallas_call(
        _edge_body,
        grid=(_GE,),
        in_specs=[
            pl.BlockSpec((16, _TE), lambda i: (0, i)),
            pl.BlockSpec((_TE, _IN), lambda i: (i, 0)),
            pl.BlockSpec((16, hw), lambda i: (0, 0)),
            pl.BlockSpec((1, hw), lambda i: (0, 0)),
            pl.BlockSpec((hw, hw), lambda i: (0, 0)),
            pl.BlockSpec((1, hw), lambda i: (0, 0)),
            pl.BlockSpec((16, hh), lambda i: (0, 0)),
            pl.BlockSpec((1, hh), lambda i: (0, 0)),
            pl.BlockSpec((hh, hh), lambda i: (0, 0)),
            pl.BlockSpec((1, hh), lambda i: (0, 0)),
        ],
        out_specs=[
            pl.BlockSpec((_TE, 16), lambda i: (i, 0)),
            pl.BlockSpec((_TE, hh), lambda i: (i, 0)),
        ],
        out_shape=[
            jax.ShapeDtypeStruct((_E, 16), _f32),
            jax.ShapeDtypeStruct((_E, hh), _f32),
        ],
    )(ea, xs, a1, b1, a2, b2, a11, b11, a21, b21)


def _agg_bn(part, cnt_col, root_w, hin, bias, g, b):
    s = part[0:_N, 0:_H] + part[_NP:_NP + _N, 0:_H]
    cnt = part[0:_N, _H:_H + 1] + part[_NP:_NP + _N, _H:_H + 1]
    agg = s / jnp.maximum(cnt, 1.0)
    h0 = agg + jnp.dot(hin, root_w, preferred_element_type=_f32) + bias
    m = jnp.mean(h0, axis=0, keepdims=True)
    v = jnp.mean((h0 - m) ** 2, axis=0, keepdims=True)
    return jnp.maximum((h0 - m) * lax.rsqrt(v + 1e-5) * g + b, 0.0)


def _node0_body(part_ref, x_ref, root_ref, bias_ref, g_ref, b_ref, out_ref):
    h = _agg_bn(part_ref[...], None, root_ref[...], x_ref[...],
                bias_ref[...], g_ref[...], b_ref[...])
    out_ref[...] = jnp.concatenate([h, jnp.zeros_like(h)], axis=1)


def _node0_call(part, x, root_w, bias, g, b):
    return pl.pallas_call(
        _node0_body,
        grid=(1,),
        in_specs=[
            pl.BlockSpec((2 * _NP, 16), lambda i: (0, 0)),
            pl.BlockSpec((_N, _IN), lambda i: (0, 0)),
            pl.BlockSpec((_IN, _H), lambda i: (0, 0)),
            pl.BlockSpec((1, _H), lambda i: (0, 0)),
            pl.BlockSpec((1, _H), lambda i: (0, 0)),
            pl.BlockSpec((1, _H), lambda i: (0, 0)),
        ],
        out_specs=pl.BlockSpec((_N, 16), lambda i: (0, 0)),
        out_shape=jax.ShapeDtypeStruct((_N, 16), _f32),
    )(part, x, root_w, bias, g, b)


_TM = 6400
_GM = _E // _TM

def _msg1_body(hs_ref, w1_ref, r8_ref, s8_ref, out_ref):
    hs = hs_ref[:, 0:_H]
    hr = jnp.dot(hs, r8_ref[...], preferred_element_type=_f32)
    msg = jnp.dot(hr * w1_ref[...], s8_ref[...], preferred_element_type=_f32)
    col = lax.broadcasted_iota(jnp.int32, (_TM, _H), 1)
    oz = jnp.where(col == 0, 1.0, 0.0).astype(_f32)
    out_ref[...] = jnp.concatenate([msg, oz], axis=1)


def _msg1_call(hs, w1, r8, s8):
    hh = _H * _H
    return pl.pallas_call(
        _msg1_body,
        grid=(_GM,),
        in_specs=[
            pl.BlockSpec((_TM, 16), lambda i: (i, 0)),
            pl.BlockSpec((_TM, hh), lambda i: (i, 0)),
            pl.BlockSpec((_H, hh), lambda i: (0, 0)),
            pl.BlockSpec((hh, _H), lambda i: (0, 0)),
        ],
        out_specs=pl.BlockSpec((_TM, 16), lambda i: (i, 0)),
        out_shape=jax.ShapeDtypeStruct((_E, 16), _f32),
    )(hs, w1, r8, s8)


def _final_body(part_ref, h_ref, root_ref, bias_ref, g_ref, b_ref,
                gw_ref, gb_ref, cw1_ref, cb1_ref, cw2_ref, cb2_ref,
                rw1_ref, rb1_ref, rw2_ref, rb2_ref, batch_ref,
                cls_ref, reg_ref):
    z = _agg_bn(part_ref[...], None, root_ref[...], h_ref[:, 0:_H],
                bias_ref[...], g_ref[...], b_ref[...])
    gate = jnp.dot(z, gw_ref[...], preferred_element_type=_f32) + gb_ref[...]
    gids = lax.broadcasted_iota(jnp.int32, (1, _G), 1)
    maskb = batch_ref[...] == gids
    maskf = maskb.astype(_f32)
    gmax = jnp.max(jnp.where(maskb, gate, -jnp.inf), axis=0, keepdims=True)
    gmax = jnp.where(jnp.isfinite(gmax), gmax, 0.0)
    gmax_n = jnp.sum(maskf * gmax, axis=1, keepdims=True)
    a = jnp.exp(gate - gmax_n)
    denom = jnp.sum(maskf * a, axis=0, keepdims=True)
    denom_n = jnp.sum(maskf * denom, axis=1, keepdims=True)
    alpha = a / (denom_n + 1e-16)
    gpool = lax.dot_general(maskf, alpha * z, (((0,), (0,)), ((), ())),
                            preferred_element_type=_f32)
    c1 = jnp.maximum(
        jnp.dot(gpool, cw1_ref[...], preferred_element_type=_f32)
        + cb1_ref[...], 0.0)
    cls_ref[...] = jnp.dot(c1, cw2_ref[...],
                           preferred_element_type=_f32) + cb2_ref[...]
    r1 = jnp.maximum(
        jnp.dot(gpool, rw1_ref[...], preferred_element_type=_f32)
        + rb1_ref[...], 0.0)
    reg_ref[...] = jnp.dot(r1, rw2_ref[...],
                           preferred_element_type=_f32) + rb2_ref[...]


def _final_call(part, h16, root_w, bias, g, b, gw, gb,
                cw1, cb1, cw2, cb2, rw1, rb1, rw2, rb2, batch_col):
    full = lambda r, c: pl.BlockSpec((r, c), lambda i: (0, 0))
    return pl.pallas_call(
        _final_body,
        grid=(1,),
        in_specs=[
            full(2 * _NP, 16),
            full(_N, 16),
            full(_H, _H), full(1, _H), full(1, _H), full(1, _H),
            full(_H, 1), full(1, 1),
            full(_H, _H), full(1, _H), full(_H, 9), full(1, 9),
            full(_H, _H), full(1, _H), full(_H, 1), full(1, 1),
            full(_N, 1),
        ],
        out_specs=[full(_G, 9), full(_G, 1)],
        out_shape=[
            jax.ShapeDtypeStruct((_G, 9), _f32),
            jax.ShapeDtypeStruct((_G, 1), _f32),
        ],
    )(part, h16, root_w, bias, g, b, gw, gb,
      cw1, cb1, cw2, cb2, rw1, rb1, rw2, rb2, batch_col)


def kernel(x, edge_attr, A1_0, b1_0, A2_0, b2_0, root0, bias0, bn_g0, bn_b0,
           A1_1, b1_1, A2_1, b2_1, root1, bias1, bn_g1, bn_b1,
           gate_w, gate_b, cls_w1, cls_b1, cls_w2, cls_b2,
           reg_w1, reg_b1, reg_w2, reg_b2, edge_index, batch):
    row = lambda t: t.reshape(1, -1)
    src3 = edge_index[0].reshape(_NW, _CH, _CW)
    dst3 = edge_index[1].reshape(_NW, _CH, _CW)
    zeros16 = jnp.zeros((_NP, 16), _f32)
    r8 = jnp.repeat(jnp.eye(_H, dtype=_f32), _H, axis=1)
    s8 = jnp.tile(jnp.eye(_H, dtype=_f32), (_H, 1))

    xs = _gather(x, src3, _IN)
    ea_t = jnp.swapaxes(edge_attr, 0, 1)
    msg0, w1 = _edge_call(ea_t, xs, A1_0, row(b1_0), A2_0, row(b2_0),
                          A1_1, row(b1_1), A2_1, row(b2_1))
    part0 = _scatter(msg0, dst3, zeros16)
    h16 = _node0_call(part0, x, root0, row(bias0), row(bn_g0), row(bn_b0))
    hs = _gather(h16, src3, 16)
    msg1 = _msg1_call(hs, w1, r8, s8)
    part1 = _scatter(msg1, dst3, zeros16)
    cls, reg = _final_call(part1, h16, root1, row(bias1), row(bn_g1),
                           row(bn_b1), gate_w, row(gate_b),
                           cls_w1, row(cls_b1), cls_w2, row(cls_b2),
                           reg_w1, row(reg_b1), reg_w2, row(reg_b2),
                           batch.reshape(-1, 1))
    return (cls, reg)


# speedup vs baseline: 2.7699x; 1.0474x over previous
import functools

import jax
import jax.numpy as jnp
from jax import lax
from jax.experimental import pallas as pl
from jax.experimental.pallas import tpu as pltpu
from jax.experimental.pallas import tpu_sc as plsc

_N = 10000
_E = 160000
_IN = 128
_H = 8
_G = 32
_NC = 2
_NS = 16
_NW = _NC * _NS
_PW = _E // _NW
_CW = 125
_CH = _PW // _CW
_CWT = 40
_CHT = _PW // _CWT
_NP = 10112
_RPS = _NP // _NS

_SC_PARAMS = pltpu.CompilerParams(use_tc_tiling_on_sc=False)

_f32 = jnp.float32
_bf16 = jnp.bfloat16


def _gather(table, idx3, d, cw, pw):
    mesh = plsc.VectorSubcoreMesh(core_axis_name="c", subcore_axis_name="s")
    ch = pw // cw

    @functools.partial(
        pl.kernel,
        out_type=jax.ShapeDtypeStruct((_NW * pw, d), _f32),
        mesh=mesh,
        compiler_params=_SC_PARAMS,
        scratch_types=[
            pltpu.VMEM((ch, cw), jnp.int32),
            pltpu.VMEM((2, cw, d), _f32),
            pltpu.SemaphoreType.DMA,
            pltpu.SemaphoreType.DMA,
        ],
    )
    def k(table_hbm, idx_hbm, out_hbm, idx_v, rows_v, gsem, ssem):
        wid = lax.axis_index("s") * _NC + lax.axis_index("c")
        pltpu.sync_copy(idx_hbm.at[wid], idx_v)
        pltpu.async_copy(table_hbm.at[idx_v.at[0]], rows_v.at[0], gsem)

        @pl.loop(0, ch)
        def _(j):
            slot = lax.rem(j, 2)
            nxt = lax.rem(j + 1, 2)
            pltpu.make_async_copy(table_hbm.at[pl.ds(0, cw)],
                                  rows_v.at[slot], gsem).wait()

            @pl.when(j > 0)
            def _():
                pltpu.make_async_copy(rows_v.at[nxt],
                                      out_hbm.at[pl.ds(0, cw)], ssem).wait()

            @pl.when(j + 1 < ch)
            def _():
                pltpu.async_copy(table_hbm.at[idx_v.at[j + 1]],
                                 rows_v.at[nxt], gsem)

            pltpu.async_copy(
                rows_v.at[slot],
                out_hbm.at[pl.ds(wid * pw + j * cw, cw)], ssem)

        pltpu.make_async_copy(rows_v.at[0],
                              out_hbm.at[pl.ds(0, cw)], ssem).wait()

    return k(table, idx3)


def _scatter(vals, idx3, zeros16, cw, pw):
    mesh = plsc.VectorSubcoreMesh(core_axis_name="c", subcore_axis_name="s")
    ch = pw // cw

    @functools.partial(
        pl.kernel,
        out_type=jax.ShapeDtypeStruct((2 * _NP, 16), _f32),
        mesh=mesh,
        compiler_params=_SC_PARAMS,
        scratch_types=[
            pltpu.VMEM((ch, cw), jnp.int32),
            pltpu.VMEM((pw, 16), _f32),
            pltpu.VMEM_SHARED((_NP, 16), _f32),
            pltpu.SemaphoreType.DMA,
        ],
    )
    def k(vals_hbm, idx_hbm, zeros_hbm, out_hbm, idx_v, vals_v, acc_sh, sem):
        cid = lax.axis_index("c")
        sid = lax.axis_index("s")
        wid = sid * _NC + cid
        pltpu.sync_copy(zeros_hbm.at[pl.ds(sid * _RPS, _RPS)],
                        acc_sh.at[pl.ds(sid * _RPS, _RPS)])
        pltpu.sync_copy(idx_hbm.at[wid], idx_v)
        pltpu.sync_copy(vals_hbm.at[pl.ds(wid * pw, pw)], vals_v)
        plsc.subcore_barrier()

        @pl.loop(0, ch)
        def _(j):
            pltpu.sync_copy(vals_v.at[pl.ds(j * cw, cw)],
                            acc_sh.at[idx_v.at[j]], add=True)

        plsc.subcore_barrier()
        pltpu.sync_copy(acc_sh.at[pl.ds(sid * _RPS, _RPS)],
                        out_hbm.at[pl.ds(cid * _NP + sid * _RPS, _RPS)])

    return k(vals, idx3, zeros16)


_EH = _E // 2
_PWH = _EH // _NW
_CHH = _PWH // _CW
_TE = 3200
_GE = _EH // _TE

_TDN = (((0,), (0,)), ((), ()))


def _edge_body(ea_ref, xs_ref, a1_ref, b1_ref, a2_ref, b2_ref,
               a11_ref, b11_ref, a21_ref, b21_ref,
               msg_ref, w1_ref):
    ea = ea_ref[...].astype(_bf16)
    h = jnp.maximum(
        lax.dot_general(ea, a1_ref[...].astype(_bf16), _TDN,
                        preferred_element_type=_f32)
        + b1_ref[...], 0.0).astype(_bf16)
    w = jnp.dot(h, a2_ref[...].astype(_bf16),
                preferred_element_type=_f32) + b2_ref[...]
    idx = lax.broadcasted_iota(jnp.int32, (_TE, _IN * _H), 1) // _H
    xr = jnp.take_along_axis(xs_ref[...], idx, axis=1)
    p = xr * w
    while p.shape[1] > _H:
        half = p.shape[1] // 2
        p = p[:, :half] + p[:, half:]
    col = lax.broadcasted_iota(jnp.int32, (_TE, _H), 1)
    oz = jnp.where(col == 0, 1.0, 0.0).astype(_f32)
    msg_ref[...] = jnp.concatenate([p, oz], axis=1)

    h1 = jnp.maximum(
        lax.dot_general(ea, a11_ref[...].astype(_bf16), _TDN,
                        preferred_element_type=_f32)
        + b11_ref[...], 0.0).astype(_bf16)
    w1_ref[...] = (jnp.dot(h1, a21_ref[...].astype(_bf16),
                           preferred_element_type=_f32)
                   + b21_ref[...]).astype(_bf16)


def _edge_call(ea, xs, a1, b1, a2, b2, a11, b11, a21, b21, off):
    hw = _IN * _H
    hh = _H * _H
    return pl.pallas_call(
        _edge_body,
        grid=(_GE,),
        in_specs=[
            pl.BlockSpec((16, _TE), lambda i, o=off: (0, i + o)),
            pl.BlockSpec((_TE, _IN), lambda i: (i, 0)),
            pl.BlockSpec((16, hw), lambda i: (0, 0)),
            pl.BlockSpec((1, hw), lambda i: (0, 0)),
            pl.BlockSpec((hw, hw), lambda i: (0, 0)),
            pl.BlockSpec((1, hw), lambda i: (0, 0)),
            pl.BlockSpec((16, hh), lambda i: (0, 0)),
            pl.BlockSpec((1, hh), lambda i: (0, 0)),
            pl.BlockSpec((hh, hh), lambda i: (0, 0)),
            pl.BlockSpec((1, hh), lambda i: (0, 0)),
        ],
        out_specs=[
            pl.BlockSpec((_TE, 16), lambda i: (i, 0)),
            pl.BlockSpec((_TE, hh), lambda i: (i, 0)),
        ],
        out_shape=[
            jax.ShapeDtypeStruct((_EH, 16), _f32),
            jax.ShapeDtypeStruct((_EH, hh), _bf16),
        ],
    )(ea, xs, a1, b1, a2, b2, a11, b11, a21, b21)


def _agg_bn(part, cnt_col, root_w, hin, bias, g, b):
    s = part[0:_N, 0:_H] + part[_NP:_NP + _N, 0:_H]
    cnt = part[0:_N, _H:_H + 1] + part[_NP:_NP + _N, _H:_H + 1]
    agg = s / jnp.maximum(cnt, 1.0)
    h0 = agg + jnp.dot(hin, root_w, preferred_element_type=_f32) + bias
    m = jnp.mean(h0, axis=0, keepdims=True)
    v = jnp.mean((h0 - m) ** 2, axis=0, keepdims=True)
    return jnp.maximum((h0 - m) * lax.rsqrt(v + 1e-5) * g + b, 0.0)


def _node0_body(part_ref, x_ref, root_ref, bias_ref, g_ref, b_ref, out_ref):
    h = _agg_bn(part_ref[...], None, root_ref[...], x_ref[...],
                bias_ref[...], g_ref[...], b_ref[...])
    out_ref[...] = jnp.concatenate([h, jnp.zeros_like(h)], axis=1)


def _node0_call(part, x, root_w, bias, g, b):
    return pl.pallas_call(
        _node0_body,
        grid=(1,),
        in_specs=[
            pl.BlockSpec((2 * _NP, 16), lambda i: (0, 0)),
            pl.BlockSpec((_N, _IN), lambda i: (0, 0)),
            pl.BlockSpec((_IN, _H), lambda i: (0, 0)),
            pl.BlockSpec((1, _H), lambda i: (0, 0)),
            pl.BlockSpec((1, _H), lambda i: (0, 0)),
            pl.BlockSpec((1, _H), lambda i: (0, 0)),
        ],
        out_specs=pl.BlockSpec((_N, 16), lambda i: (0, 0)),
        out_shape=jax.ShapeDtypeStruct((_N, 16), _f32),
    )(part, x, root_w, bias, g, b)


_TM = 8000
_GM = _EH // _TM

def _msg1_body(hs_ref, w1_ref, r8_ref, s8_ref, out_ref):
    hs = hs_ref[:, 0:_H]
    hr = jnp.dot(hs, r8_ref[...], preferred_element_type=_f32)
    msg = jnp.dot(hr * w1_ref[...].astype(_f32), s8_ref[...],
                  preferred_element_type=_f32)
    col = lax.broadcasted_iota(jnp.int32, (_TM, _H), 1)
    oz = jnp.where(col == 0, 1.0, 0.0).astype(_f32)
    out_ref[...] = jnp.concatenate([msg, oz], axis=1)


def _msg1_call(hs, w1, r8, s8):
    hh = _H * _H
    return pl.pallas_call(
        _msg1_body,
        grid=(_GM,),
        in_specs=[
            pl.BlockSpec((_TM, 16), lambda i: (i, 0)),
            pl.BlockSpec((_TM, hh), lambda i: (i, 0)),
            pl.BlockSpec((_H, hh), lambda i: (0, 0)),
            pl.BlockSpec((hh, _H), lambda i: (0, 0)),
        ],
        out_specs=pl.BlockSpec((_TM, 16), lambda i: (i, 0)),
        out_shape=jax.ShapeDtypeStruct((_EH, 16), _f32),
    )(hs, w1, r8, s8)


def _final_body(part_ref, h_ref, root_ref, bias_ref, g_ref, b_ref,
                gw_ref, gb_ref, cw1_ref, cb1_ref, cw2_ref, cb2_ref,
                rw1_ref, rb1_ref, rw2_ref, rb2_ref, batch_ref,
                cls_ref, reg_ref):
    z = _agg_bn(part_ref[...], None, root_ref[...], h_ref[:, 0:_H],
                bias_ref[...], g_ref[...], b_ref[...])
    gate = jnp.dot(z, gw_ref[...], preferred_element_type=_f32) + gb_ref[...]
    gids = lax.broadcasted_iota(jnp.int32, (1, _G), 1)
    maskb = batch_ref[...] == gids
    maskf = maskb.astype(_f32)
    gmax = jnp.max(jnp.where(maskb, gate, -jnp.inf), axis=0, keepdims=True)
    gmax = jnp.where(jnp.isfinite(gmax), gmax, 0.0)
    gmax_n = jnp.sum(maskf * gmax, axis=1, keepdims=True)
    a = jnp.exp(gate - gmax_n)
    denom = jnp.sum(maskf * a, axis=0, keepdims=True)
    denom_n = jnp.sum(maskf * denom, axis=1, keepdims=True)
    alpha = a / (denom_n + 1e-16)
    gpool = lax.dot_general(maskf, alpha * z, (((0,), (0,)), ((), ())),
                            preferred_element_type=_f32)
    c1 = jnp.maximum(
        jnp.dot(gpool, cw1_ref[...], preferred_element_type=_f32)
        + cb1_ref[...], 0.0)
    cls_ref[...] = jnp.dot(c1, cw2_ref[...],
                           preferred_element_type=_f32) + cb2_ref[...]
    r1 = jnp.maximum(
        jnp.dot(gpool, rw1_ref[...], preferred_element_type=_f32)
        + rb1_ref[...], 0.0)
    reg_ref[...] = jnp.dot(r1, rw2_ref[...],
                           preferred_element_type=_f32) + rb2_ref[...]


def _final_call(part, h16, root_w, bias, g, b, gw, gb,
                cw1, cb1, cw2, cb2, rw1, rb1, rw2, rb2, batch_col):
    full = lambda r, c: pl.BlockSpec((r, c), lambda i: (0, 0))
    return pl.pallas_call(
        _final_body,
        grid=(1,),
        in_specs=[
            full(2 * _NP, 16),
            full(_N, 16),
            full(_H, _H), full(1, _H), full(1, _H), full(1, _H),
            full(_H, 1), full(1, 1),
            full(_H, _H), full(1, _H), full(_H, 9), full(1, 9),
            full(_H, _H), full(1, _H), full(_H, 1), full(1, 1),
            full(_N, 1),
        ],
        out_specs=[full(_G, 9), full(_G, 1)],
        out_shape=[
            jax.ShapeDtypeStruct((_G, 9), _f32),
            jax.ShapeDtypeStruct((_G, 1), _f32),
        ],
    )(part, h16, root_w, bias, g, b, gw, gb,
      cw1, cb1, cw2, cb2, rw1, rb1, rw2, rb2, batch_col)


def kernel(x, edge_attr, A1_0, b1_0, A2_0, b2_0, root0, bias0, bn_g0, bn_b0,
           A1_1, b1_1, A2_1, b2_1, root1, bias1, bn_g1, bn_b1,
           gate_w, gate_b, cls_w1, cls_b1, cls_w2, cls_b2,
           reg_w1, reg_b1, reg_w2, reg_b2, edge_index, batch):
    row = lambda t: t.reshape(1, -1)
    src = [edge_index[0, o * _EH:(o + 1) * _EH].reshape(_NW, _CHH, _CW)
           for o in (0, 1)]
    dst = [edge_index[1, o * _EH:(o + 1) * _EH].reshape(_NW, _CHH, _CW)
           for o in (0, 1)]
    zeros16 = jnp.zeros((_NP, 16), _f32)
    r8 = jnp.repeat(jnp.eye(_H, dtype=_f32), _H, axis=1)
    s8 = jnp.tile(jnp.eye(_H, dtype=_f32), (_H, 1))
    ea_t = jnp.swapaxes(edge_attr, 0, 1)

    xs = [_gather(x, src[o], _IN, _CW, _PWH) for o in (0, 1)]
    ew = [_edge_call(ea_t, xs[o], A1_0, row(b1_0), A2_0, row(b2_0),
                     A1_1, row(b1_1), A2_1, row(b2_1), o * _GE)
          for o in (0, 1)]
    part0 = (_scatter(ew[0][0], dst[0], zeros16, _CW, _PWH)
             + _scatter(ew[1][0], dst[1], zeros16, _CW, _PWH))
    h16 = _node0_call(part0, x, root0, row(bias0), row(bn_g0), row(bn_b0))
    hs = [_gather(h16, src[o], 16, _CW, _PWH) for o in (0, 1)]
    msg1 = [_msg1_call(hs[o], ew[o][1], r8, s8) for o in (0, 1)]
    part1 = (_scatter(msg1[0], dst[0], zeros16, _CW, _PWH)
             + _scatter(msg1[1], dst[1], zeros16, _CW, _PWH))
    cls, reg = _final_call(part1, h16, root1, row(bias1), row(bn_g1),
                           row(bn_b1), gate_w, row(gate_b),
                           cls_w1, row(cls_b1), cls_w2, row(cls_b2),
                           reg_w1, row(reg_b1), reg_w2, row(reg_b2),
                           batch.reshape(-1, 1))
    return (cls, reg)


# speedup vs baseline: 2.8404x; 1.0254x over previous
import functools

import jax
import jax.numpy as jnp
from jax import lax
from jax.experimental import pallas as pl
from jax.experimental.pallas import tpu as pltpu
from jax.experimental.pallas import tpu_sc as plsc

_N = 10000
_E = 160000
_IN = 128
_H = 8
_G = 32
_NC = 2
_NS = 16
_NW = _NC * _NS
_PW = _E // _NW
_CW = 125
_CH = _PW // _CW
_CWT = 40
_CHT = _PW // _CWT
_NP = 10112
_RPS = _NP // _NS

_SC_PARAMS = pltpu.CompilerParams(use_tc_tiling_on_sc=False)

_f32 = jnp.float32
_bf16 = jnp.bfloat16


def _gather(table, idx3, d, cw, pw):
    mesh = plsc.VectorSubcoreMesh(core_axis_name="c", subcore_axis_name="s")
    ch = pw // cw

    @functools.partial(
        pl.kernel,
        out_type=jax.ShapeDtypeStruct((_NW * pw, d), _f32),
        mesh=mesh,
        compiler_params=_SC_PARAMS,
        scratch_types=[
            pltpu.VMEM((ch, cw), jnp.int32),
            pltpu.VMEM((2, cw, d), _f32),
            pltpu.SemaphoreType.DMA,
            pltpu.SemaphoreType.DMA,
        ],
    )
    def k(table_hbm, idx_hbm, out_hbm, idx_v, rows_v, gsem, ssem):
        wid = lax.axis_index("s") * _NC + lax.axis_index("c")
        pltpu.sync_copy(idx_hbm.at[wid], idx_v)
        pltpu.async_copy(table_hbm.at[idx_v.at[0]], rows_v.at[0], gsem)

        @pl.loop(0, ch)
        def _(j):
            slot = lax.rem(j, 2)
            nxt = lax.rem(j + 1, 2)
            pltpu.make_async_copy(table_hbm.at[pl.ds(0, cw)],
                                  rows_v.at[slot], gsem).wait()

            @pl.when(j > 0)
            def _():
                pltpu.make_async_copy(rows_v.at[nxt],
                                      out_hbm.at[pl.ds(0, cw)], ssem).wait()

            @pl.when(j + 1 < ch)
            def _():
                pltpu.async_copy(table_hbm.at[idx_v.at[j + 1]],
                                 rows_v.at[nxt], gsem)

            pltpu.async_copy(
                rows_v.at[slot],
                out_hbm.at[pl.ds(wid * pw + j * cw, cw)], ssem)

        pltpu.make_async_copy(rows_v.at[0],
                              out_hbm.at[pl.ds(0, cw)], ssem).wait()

    return k(table, idx3)


def _scatter(vals, idx3, zeros16, cw, pw):
    mesh = plsc.VectorSubcoreMesh(core_axis_name="c", subcore_axis_name="s")
    ch = pw // cw

    @functools.partial(
        pl.kernel,
        out_type=jax.ShapeDtypeStruct((2 * _NP, 16), _f32),
        mesh=mesh,
        compiler_params=_SC_PARAMS,
        scratch_types=[
            pltpu.VMEM((ch, cw), jnp.int32),
            pltpu.VMEM((pw, 16), _f32),
            pltpu.VMEM_SHARED((_NP, 16), _f32),
            pltpu.SemaphoreType.DMA,
        ],
    )
    def k(vals_hbm, idx_hbm, zeros_hbm, out_hbm, idx_v, vals_v, acc_sh, sem):
        cid = lax.axis_index("c")
        sid = lax.axis_index("s")
        wid = sid * _NC + cid
        pltpu.sync_copy(zeros_hbm.at[pl.ds(sid * _RPS, _RPS)],
                        acc_sh.at[pl.ds(sid * _RPS, _RPS)])
        pltpu.sync_copy(idx_hbm.at[wid], idx_v)
        pltpu.sync_copy(vals_hbm.at[pl.ds(wid * pw, pw)], vals_v)
        plsc.subcore_barrier()

        @pl.loop(0, ch)
        def _(j):
            pltpu.sync_copy(vals_v.at[pl.ds(j * cw, cw)],
                            acc_sh.at[idx_v.at[j]], add=True)

        plsc.subcore_barrier()
        pltpu.sync_copy(acc_sh.at[pl.ds(sid * _RPS, _RPS)],
                        out_hbm.at[pl.ds(cid * _NP + sid * _RPS, _RPS)])

    return k(vals, idx3, zeros16)


_EH = _E // 2
_PWH = _EH // _NW
_CHH = _PWH // _CW
_TE = 3200
_GE = _EH // _TE

_TDN = (((0,), (0,)), ((), ()))


def _edge_body(ea_ref, xs_ref, a1_ref, b1_ref, a2_ref, b2_ref,
               a11_ref, b11_ref, a21_ref, b21_ref,
               msg_ref, w1_ref):
    ea = ea_ref[...].astype(_bf16)
    h = jnp.maximum(
        lax.dot_general(ea, a1_ref[...].astype(_bf16), _TDN,
                        preferred_element_type=_f32)
        + b1_ref[...], 0.0).astype(_bf16)
    w = jnp.dot(h, a2_ref[...].astype(_bf16),
                preferred_element_type=_f32) + b2_ref[...]
    idx = lax.broadcasted_iota(jnp.int32, (_TE, _IN * _H), 1) // _H
    xr = jnp.take_along_axis(xs_ref[...], idx, axis=1)
    p = xr * w
    while p.shape[1] > _H:
        half = p.shape[1] // 2
        p = p[:, :half] + p[:, half:]
    col = lax.broadcasted_iota(jnp.int32, (_TE, _H), 1)
    oz = jnp.where(col == 0, 1.0, 0.0).astype(_f32)
    msg_ref[...] = jnp.concatenate([p, oz], axis=1)

    h1 = jnp.maximum(
        lax.dot_general(ea, a11_ref[...].astype(_bf16), _TDN,
                        preferred_element_type=_f32)
        + b11_ref[...], 0.0).astype(_bf16)
    w1_ref[...] = (jnp.dot(h1, a21_ref[...].astype(_bf16),
                           preferred_element_type=_f32)
                   + b21_ref[...]).astype(_bf16)


def _edge_call(ea, xs, a1, b1, a2, b2, a11, b11, a21, b21, off):
    hw = _IN * _H
    hh = _H * _H
    return pl.pallas_call(
        _edge_body,
        grid=(_GE,),
        in_specs=[
            pl.BlockSpec((16, _TE), lambda i, o=off: (0, i + o)),
            pl.BlockSpec((_TE, _IN), lambda i: (i, 0)),
            pl.BlockSpec((16, hw), lambda i: (0, 0)),
            pl.BlockSpec((1, hw), lambda i: (0, 0)),
            pl.BlockSpec((hw, hw), lambda i: (0, 0)),
            pl.BlockSpec((1, hw), lambda i: (0, 0)),
            pl.BlockSpec((16, hh), lambda i: (0, 0)),
            pl.BlockSpec((1, hh), lambda i: (0, 0)),
            pl.BlockSpec((hh, hh), lambda i: (0, 0)),
            pl.BlockSpec((1, hh), lambda i: (0, 0)),
        ],
        out_specs=[
            pl.BlockSpec((_TE, 16), lambda i: (i, 0)),
            pl.BlockSpec((_TE, hh), lambda i: (i, 0)),
        ],
        out_shape=[
            jax.ShapeDtypeStruct((_EH, 16), _f32),
            jax.ShapeDtypeStruct((_EH, hh), _bf16),
        ],
    )(ea, xs, a1, b1, a2, b2, a11, b11, a21, b21)


def _agg_bn(pa, pb, root_w, hin, bias, g, b):
    s = (pa[0:_N, 0:_H] + pa[_NP:_NP + _N, 0:_H]
         + pb[0:_N, 0:_H] + pb[_NP:_NP + _N, 0:_H])
    cnt = (pa[0:_N, _H:_H + 1] + pa[_NP:_NP + _N, _H:_H + 1]
           + pb[0:_N, _H:_H + 1] + pb[_NP:_NP + _N, _H:_H + 1])
    agg = s / jnp.maximum(cnt, 1.0)
    h0 = agg + jnp.dot(hin, root_w, preferred_element_type=_f32) + bias
    m = jnp.mean(h0, axis=0, keepdims=True)
    v = jnp.mean((h0 - m) ** 2, axis=0, keepdims=True)
    return jnp.maximum((h0 - m) * lax.rsqrt(v + 1e-5) * g + b, 0.0)


def _node0_body(pa_ref, pb_ref, x_ref, root_ref, bias_ref, g_ref, b_ref,
                r8_ref, out_ref, hx_ref):
    h = _agg_bn(pa_ref[...], pb_ref[...], root_ref[...], x_ref[...],
                bias_ref[...], g_ref[...], b_ref[...])
    out_ref[...] = jnp.concatenate([h, jnp.zeros_like(h)], axis=1)
    hx = jnp.dot(h, r8_ref[...], preferred_element_type=_f32)
    hx_ref[...] = jnp.concatenate([hx, jnp.zeros_like(hx)], axis=1)


def _node0_call(pa, pb, x, root_w, bias, g, b, r8):
    return pl.pallas_call(
        _node0_body,
        grid=(1,),
        in_specs=[
            pl.BlockSpec((2 * _NP, 16), lambda i: (0, 0)),
            pl.BlockSpec((2 * _NP, 16), lambda i: (0, 0)),
            pl.BlockSpec((_N, _IN), lambda i: (0, 0)),
            pl.BlockSpec((_IN, _H), lambda i: (0, 0)),
            pl.BlockSpec((1, _H), lambda i: (0, 0)),
            pl.BlockSpec((1, _H), lambda i: (0, 0)),
            pl.BlockSpec((1, _H), lambda i: (0, 0)),
            pl.BlockSpec((_H, _H * _H), lambda i: (0, 0)),
        ],
        out_specs=[
            pl.BlockSpec((_N, 16), lambda i: (0, 0)),
            pl.BlockSpec((_N, _IN), lambda i: (0, 0)),
        ],
        out_shape=[
            jax.ShapeDtypeStruct((_N, 16), _f32),
            jax.ShapeDtypeStruct((_N, _IN), _f32),
        ],
    )(pa, pb, x, root_w, bias, g, b, r8)


_TM = 8000
_GM = _EH // _TM

def _msg1_body(hx_ref, w1_ref, s8_ref, out_ref):
    hr = hx_ref[:, 0:_H * _H]
    msg = jnp.dot(hr * w1_ref[...].astype(_f32), s8_ref[...],
                  preferred_element_type=_f32)
    col = lax.broadcasted_iota(jnp.int32, (_TM, _H), 1)
    oz = jnp.where(col == 0, 1.0, 0.0).astype(_f32)
    out_ref[...] = jnp.concatenate([msg, oz], axis=1)


def _msg1_call(hx, w1, s8):
    hh = _H * _H
    return pl.pallas_call(
        _msg1_body,
        grid=(_GM,),
        in_specs=[
            pl.BlockSpec((_TM, _IN), lambda i: (i, 0)),
            pl.BlockSpec((_TM, hh), lambda i: (i, 0)),
            pl.BlockSpec((hh, _H), lambda i: (0, 0)),
        ],
        out_specs=pl.BlockSpec((_TM, 16), lambda i: (i, 0)),
        out_shape=jax.ShapeDtypeStruct((_EH, 16), _f32),
    )(hx, w1, s8)


def _final_body(pa_ref, pb_ref, h_ref, root_ref, bias_ref, g_ref, b_ref,
                gw_ref, gb_ref, cw1_ref, cb1_ref, cw2_ref, cb2_ref,
                rw1_ref, rb1_ref, rw2_ref, rb2_ref, batch_ref,
                cls_ref, reg_ref):
    z = _agg_bn(pa_ref[...], pb_ref[...], root_ref[...], h_ref[:, 0:_H],
                bias_ref[...], g_ref[...], b_ref[...])
    gate = jnp.dot(z, gw_ref[...], preferred_element_type=_f32) + gb_ref[...]
    gids = lax.broadcasted_iota(jnp.int32, (1, _G), 1)
    maskb = batch_ref[...] == gids
    maskf = maskb.astype(_f32)
    gmax = jnp.max(jnp.where(maskb, gate, -jnp.inf), axis=0, keepdims=True)
    gmax = jnp.where(jnp.isfinite(gmax), gmax, 0.0)
    gmax_n = jnp.sum(maskf * gmax, axis=1, keepdims=True)
    a = jnp.exp(gate - gmax_n)
    denom = jnp.sum(maskf * a, axis=0, keepdims=True)
    denom_n = jnp.sum(maskf * denom, axis=1, keepdims=True)
    alpha = a / (denom_n + 1e-16)
    gpool = lax.dot_general(maskf, alpha * z, (((0,), (0,)), ((), ())),
                            preferred_element_type=_f32)
    c1 = jnp.maximum(
        jnp.dot(gpool, cw1_ref[...], preferred_element_type=_f32)
        + cb1_ref[...], 0.0)
    cls_ref[...] = jnp.dot(c1, cw2_ref[...],
                           preferred_element_type=_f32) + cb2_ref[...]
    r1 = jnp.maximum(
        jnp.dot(gpool, rw1_ref[...], preferred_element_type=_f32)
        + rb1_ref[...], 0.0)
    reg_ref[...] = jnp.dot(r1, rw2_ref[...],
                           preferred_element_type=_f32) + rb2_ref[...]


def _final_call(pa, pb, h16, root_w, bias, g, b, gw, gb,
                cw1, cb1, cw2, cb2, rw1, rb1, rw2, rb2, batch_col):
    full = lambda r, c: pl.BlockSpec((r, c), lambda i: (0, 0))
    return pl.pallas_call(
        _final_body,
        grid=(1,),
        in_specs=[
            full(2 * _NP, 16),
            full(2 * _NP, 16),
            full(_N, 16),
            full(_H, _H), full(1, _H), full(1, _H), full(1, _H),
            full(_H, 1), full(1, 1),
            full(_H, _H), full(1, _H), full(_H, 9), full(1, 9),
            full(_H, _H), full(1, _H), full(_H, 1), full(1, 1),
            full(_N, 1),
        ],
        out_specs=[full(_G, 9), full(_G, 1)],
        out_shape=[
            jax.ShapeDtypeStruct((_G, 9), _f32),
            jax.ShapeDtypeStruct((_G, 1), _f32),
        ],
    )(pa, pb, h16, root_w, bias, g, b, gw, gb,
      cw1, cb1, cw2, cb2, rw1, rb1, rw2, rb2, batch_col)


def kernel(x, edge_attr, A1_0, b1_0, A2_0, b2_0, root0, bias0, bn_g0, bn_b0,
           A1_1, b1_1, A2_1, b2_1, root1, bias1, bn_g1, bn_b1,
           gate_w, gate_b, cls_w1, cls_b1, cls_w2, cls_b2,
           reg_w1, reg_b1, reg_w2, reg_b2, edge_index, batch):
    row = lambda t: t.reshape(1, -1)
    src = [edge_index[0, o * _EH:(o + 1) * _EH].reshape(_NW, _CHH, _CW)
           for o in (0, 1)]
    dst = [edge_index[1, o * _EH:(o + 1) * _EH].reshape(_NW, _CHH, _CW)
           for o in (0, 1)]
    zeros16 = jnp.zeros((_NP, 16), _f32)
    r8 = jnp.repeat(jnp.eye(_H, dtype=_f32), _H, axis=1)
    s8 = jnp.tile(jnp.eye(_H, dtype=_f32), (_H, 1))
    ea_t = jnp.swapaxes(edge_attr, 0, 1)

    xs = [_gather(x, src[o], _IN, _CW, _PWH) for o in (0, 1)]
    ew = [_edge_call(ea_t, xs[o], A1_0, row(b1_0), A2_0, row(b2_0),
                     A1_1, row(b1_1), A2_1, row(b2_1), o * _GE)
          for o in (0, 1)]
    part0a = _scatter(ew[0][0], dst[0], zeros16, _CW, _PWH)
    part0b = _scatter(ew[1][0], dst[1], zeros16, _CW, _PWH)
    h16, hx = _node0_call(part0a, part0b, x, root0, row(bias0),
                          row(bn_g0), row(bn_b0), r8)
    hxs = [_gather(hx, src[o], _IN, _CW, _PWH) for o in (0, 1)]
    msg1 = [_msg1_call(hxs[o], ew[o][1], s8) for o in (0, 1)]
    part1a = _scatter(msg1[0], dst[0], zeros16, _CW, _PWH)
    part1b = _scatter(msg1[1], dst[1], zeros16, _CW, _PWH)
    cls, reg = _final_call(part1a, part1b, h16, root1, row(bias1), row(bn_g1),
                           row(bn_b1), gate_w, row(gate_b),
                           cls_w1, row(cls_b1), cls_w2, row(cls_b2),
                           reg_w1, row(reg_b1), reg_w2, row(reg_b2),
                           batch.reshape(-1, 1))
    return (cls, reg)
